# baseline (speedup 1.0000x reference)
import functools

import jax
import jax.numpy as jnp
from jax import lax
from jax.experimental import pallas as pl
from jax.experimental.pallas import tpu as pltpu
from jax.experimental.pallas import tpu_sc as plsc

B = 32
SPLIT = 14
NOISE_STD = 0.05


def _t_gather_kernel(t_epl_hbm, idx_hbm, t_out, tab_s, idx_s, t_s):
    cid = lax.axis_index("c")

    @pl.when(cid == 0)
    def _():
        pltpu.sync_copy(t_epl_hbm, tab_s)
        pltpu.sync_copy(idx_hbm, idx_s)
        for i in range(B):
            t_s[i] = tab_s[idx_s[i]]
        pltpu.sync_copy(t_s, t_out)


def _gather_t(t_epl, random_indices):
    mesh = plsc.ScalarSubcoreMesh(axis_name="c", num_cores=1)
    kern = functools.partial(
        pl.kernel,
        mesh=mesh,
        out_type=jax.ShapeDtypeStruct((B,), jnp.int32),
        scratch_types=[
            pltpu.SMEM((64,), jnp.int32),
            pltpu.SMEM((B,), jnp.int32),
            pltpu.SMEM((B,), jnp.int32),
        ],
    )(_t_gather_kernel)
    return kern(t_epl, random_indices)


def _combine_a_kernel(idx_ref, t_epl_ref, atab_ref, btab_ref, x_ref, n_ref, o_ref):
    i = pl.program_id(0)
    for j in range(2):
        t = t_epl_ref[idx_ref[2 * i + j]]
        a = atab_ref[t]
        b = btab_ref[t] * NOISE_STD
        o_ref[j] = a * x_ref[j] + b * n_ref[j]


def _combine_a(idx, t_epl, atab, btab, x, n):
    _, C, H, W = x.shape
    return pl.pallas_call(
        _combine_a_kernel,
        grid=(SPLIT,),
        in_specs=[
            pl.BlockSpec(memory_space=pltpu.SMEM),
            pl.BlockSpec(memory_space=pltpu.SMEM),
            pl.BlockSpec(memory_space=pltpu.SMEM),
            pl.BlockSpec(memory_space=pltpu.SMEM),
            pl.BlockSpec((2, C, H, W), lambda i: (i, 0, 0, 0)),
            pl.BlockSpec((2, C, H, W), lambda i: (i, 0, 0, 0)),
        ],
        out_specs=pl.BlockSpec((2, C, H, W), lambda i: (i, 0, 0, 0)),
        out_shape=jax.ShapeDtypeStruct(x.shape, jnp.float32),
    )(idx, t_epl, atab, btab, x, n)


def _combine_b_kernel(idx_ref, t_epl_ref, atab_ref, btab_ref, x_ref, n_ref,
                      prev_ref, o_ref):
    i = pl.program_id(0)
    for j in range(2):
        t = t_epl_ref[idx_ref[2 * (i + SPLIT) + j]]
        a = atab_ref[t]
        b = btab_ref[t] * NOISE_STD
        o_ref[j] = a * x_ref[j] + b * n_ref[j]


def _combine_b(idx, t_epl, atab, btab, x, n, prev):
    _, C, H, W = x.shape
    return pl.pallas_call(
        _combine_b_kernel,
        grid=(B // 2 - SPLIT,),
        in_specs=[
            pl.BlockSpec(memory_space=pltpu.SMEM),
            pl.BlockSpec(memory_space=pltpu.SMEM),
            pl.BlockSpec(memory_space=pltpu.SMEM),
            pl.BlockSpec(memory_space=pltpu.SMEM),
            pl.BlockSpec((2, C, H, W), lambda i: (i + SPLIT, 0, 0, 0)),
            pl.BlockSpec((2, C, H, W), lambda i: (i + SPLIT, 0, 0, 0)),
            pl.BlockSpec(memory_space=pl.ANY),
        ],
        out_specs=pl.BlockSpec((2, C, H, W), lambda i: (i + SPLIT, 0, 0, 0)),
        out_shape=jax.ShapeDtypeStruct(prev.shape, jnp.float32),
        input_output_aliases={6: 0},
    )(idx, t_epl, atab, btab, x, n, prev)


def kernel(x_0, alphas_bar_sqrt, one_minus_alphas_bar_sqrt, t_epl, random_indices, noise):
    out_a = _combine_a(random_indices, t_epl, alphas_bar_sqrt,
                       one_minus_alphas_bar_sqrt, x_0, noise)
    t_epl_d, out_a = lax.optimization_barrier((t_epl, out_a))
    t = _gather_t(t_epl_d, random_indices)
    out = _combine_b(random_indices, t_epl, alphas_bar_sqrt,
                     one_minus_alphas_bar_sqrt, x_0, noise, out_a)
    return (out, t.reshape(-1, 1))


# speedup vs baseline: 1.0153x; 1.0153x over previous
import functools

import jax
import jax.numpy as jnp
from jax import lax
from jax.experimental import pallas as pl
from jax.experimental.pallas import tpu as pltpu
from jax.experimental.pallas import tpu_sc as plsc

B = 32
NOISE_STD = 0.05


def _t_gather_kernel(t_epl_hbm, idx_hbm, t_out, idx_v, t_v, sem):
    cid = lax.axis_index("c")
    sid = lax.axis_index("s")

    @pl.when(jnp.logical_and(cid == 0, sid == 0))
    def _():
        pltpu.sync_copy(idx_hbm, idx_v)
        pltpu.async_copy(t_epl_hbm.at[idx_v], t_v, sem).wait()
        pltpu.sync_copy(t_v, t_out)


def _gather_t(t_epl, random_indices):
    mesh = plsc.VectorSubcoreMesh(core_axis_name="c", subcore_axis_name="s",
                                  num_cores=1)
    kern = functools.partial(
        pl.kernel,
        mesh=mesh,
        out_type=jax.ShapeDtypeStruct((B,), jnp.int32),
        scratch_types=[
            pltpu.VMEM((B,), jnp.int32),
            pltpu.VMEM((B,), jnp.int32),
            pltpu.SemaphoreType.DMA,
        ],
    )(_t_gather_kernel)
    return kern(t_epl, random_indices)


def _combine_kernel(idx_ref, t_epl_ref, atab_ref, btab_ref, x_ref, n_ref, o_ref):
    i = pl.program_id(0)
    for j in range(2):
        t = t_epl_ref[idx_ref[2 * i + j]]
        a = atab_ref[t]
        b = btab_ref[t] * NOISE_STD
        o_ref[j] = a * x_ref[j] + b * n_ref[j]


def _combine(idx, t_epl, atab, btab, x, n):
    _, C, H, W = x.shape
    return pl.pallas_call(
        _combine_kernel,
        grid=(B // 2,),
        in_specs=[
            pl.BlockSpec(memory_space=pltpu.SMEM),
            pl.BlockSpec(memory_space=pltpu.SMEM),
            pl.BlockSpec(memory_space=pltpu.SMEM),
            pl.BlockSpec(memory_space=pltpu.SMEM),
            pl.BlockSpec((2, C, H, W), lambda i: (i, 0, 0, 0)),
            pl.BlockSpec((2, C, H, W), lambda i: (i, 0, 0, 0)),
        ],
        out_specs=pl.BlockSpec((2, C, H, W), lambda i: (i, 0, 0, 0)),
        out_shape=jax.ShapeDtypeStruct(x.shape, jnp.float32),
    )(idx, t_epl, atab, btab, x, n)


def kernel(x_0, alphas_bar_sqrt, one_minus_alphas_bar_sqrt, t_epl, random_indices, noise):
    t = _gather_t(t_epl, random_indices)
    out = _combine(random_indices, t_epl, alphas_bar_sqrt,
                   one_minus_alphas_bar_sqrt, x_0, noise)
    return (out, t.reshape(-1, 1))
